# baseline (speedup 1.0000x reference)
import functools

import jax
import jax.numpy as jnp
from jax import lax
from jax.experimental import pallas as pl
from jax.experimental.pallas import tpu as pltpu
from jax.experimental.pallas import tpu_sc as plsc

_N = 10000
_E = 320000
_H = 128
_OUT = 128
_G = 64
_NZ = 100

_NC = 2
_NS = 16
_NW = _NC * _NS
_K = 128
_NPAD = -(-_N // (_NW * 8)) * (_NW * 8)
_STRIPE = _NPAD // _NW

_SCH = 2048
_NSC = -(-_E // _SCH)
_ESCH = _NSC * _SCH
_SENT = 0x3FFFFFFF
_FLUSH = 2048
_GRP = 8
_TAIL = _FLUSH + _GRP * _K
_BUF = 4160
_CAP2 = 323072

_PREC = lax.Precision.HIGHEST


def _bucket_edges(scan_src, scan_dst):
    @functools.partial(
        pl.kernel,
        out_type=(jax.ShapeDtypeStruct((_NW * _CAP2,), jnp.int32),
                  jax.ShapeDtypeStruct((_NW * _CAP2,), jnp.int32),
                  jax.ShapeDtypeStruct((_NW * 16,), jnp.int32)),
        mesh=plsc.VectorSubcoreMesh(core_axis_name="c", subcore_axis_name="s"),
        compiler_params=pltpu.CompilerParams(needs_layout_passes=False),
        scratch_types=[
            pltpu.VMEM((_SCH,), jnp.int32),
            pltpu.VMEM((_SCH,), jnp.int32),
            pltpu.VMEM((_BUF,), jnp.int32),
            pltpu.VMEM((_BUF,), jnp.int32),
            pltpu.VMEM((16,), jnp.int32),
        ],
    )
    def body(ss_hbm, sd_hbm, ls_hbm, ld_hbm, cnt_hbm,
             scan_s, scan_d, buf_s, buf_d, cnt_v):
        cid = lax.axis_index("c")
        sid = lax.axis_index("s")
        w = cid * _NS + sid
        lo = w * _STRIPE
        hi = lo + _STRIPE
        base = w * _CAP2

        def scan_chunk(ch, carry):
            fill, cnt = carry
            e0 = ch * _SCH
            pltpu.sync_copy(ss_hbm.at[pl.ds(pl.multiple_of(e0, _SCH), _SCH)],
                            scan_s)
            pltpu.sync_copy(sd_hbm.at[pl.ds(pl.multiple_of(e0, _SCH), _SCH)],
                            scan_d)

            def vec(i, fill):
                vs = scan_s[pl.ds(i * 16, 16)]
                vd = scan_d[pl.ds(i * 16, 16)]
                m = jnp.logical_and(vd >= lo, vd < hi)
                mi = jnp.where(m, jnp.int32(1), jnp.int32(0))
                incl = plsc.cumsum(mi)
                idx = (incl - mi) + fill
                plsc.store_scatter(buf_s, [idx], vs, mask=m)
                plsc.store_scatter(buf_d, [idx], vd, mask=m)
                return fill + incl[15]

            fill = lax.fori_loop(0, _SCH // 16, vec, fill, unroll=4)
            do_flush = fill >= _FLUSH

            @pl.when(do_flush)
            def _():
                off = pl.multiple_of(base + cnt, _K)
                pltpu.sync_copy(buf_s.at[pl.ds(0, _FLUSH)],
                                ls_hbm.at[pl.ds(off, _FLUSH)])
                pltpu.sync_copy(buf_d.at[pl.ds(0, _FLUSH)],
                                ld_hbm.at[pl.ds(off, _FLUSH)])
                rem = fill - _FLUSH

                def mv(k, c):
                    buf_s[pl.ds(k * 16, 16)] = buf_s[pl.ds(_FLUSH + k * 16, 16)]
                    buf_d[pl.ds(k * 16, 16)] = buf_d[pl.ds(_FLUSH + k * 16, 16)]
                    return c

                lax.fori_loop(0, (rem + 15) // 16, mv, 0)

            fill = jnp.where(do_flush, fill - _FLUSH, fill)
            cnt = jnp.where(do_flush, cnt + _FLUSH, cnt)
            return fill, cnt

        fill, cnt = lax.fori_loop(0, _NSC, scan_chunk,
                                  (jnp.int32(0), jnp.int32(0)))

        zvec = jnp.zeros((16,), jnp.int32)
        dvec = jnp.full((16,), _N, jnp.int32)
        for t in range(_GRP * _K // 16 + 1):
            buf_s[pl.ds(fill + t * 16, 16)] = zvec
            buf_d[pl.ds(fill + t * 16, 16)] = dvec
        off = pl.multiple_of(base + cnt, _K)
        pltpu.sync_copy(buf_s.at[pl.ds(0, _TAIL)], ls_hbm.at[pl.ds(off, _TAIL)])
        pltpu.sync_copy(buf_d.at[pl.ds(0, _TAIL)], ld_hbm.at[pl.ds(off, _TAIL)])

        gk = _GRP * _K
        n_ch = cnt // _K + _GRP * ((fill + gk - 1) // gk)
        cnt_v[...] = jnp.full((16,), 1, jnp.int32) * n_ch
        pltpu.sync_copy(cnt_v,
                        cnt_hbm.at[pl.ds(pl.multiple_of(w * 16, 16), 16)])

    return body(scan_src, scan_dst)


def _edge_segment_sum(h, lists_src, lists_dst, counts, zeros):
    ls3 = lists_src.reshape(_NW, _CAP2 // _K, _K)
    ld3 = lists_dst.reshape(_NW, _CAP2 // _K, _K)

    @functools.partial(
        pl.kernel,
        out_type=jax.ShapeDtypeStruct((_NPAD, _H), jnp.float32),
        mesh=plsc.VectorSubcoreMesh(core_axis_name="c", subcore_axis_name="s"),
        compiler_params=pltpu.CompilerParams(needs_layout_passes=False),
        scratch_types=[
            pltpu.VMEM((16,), jnp.int32),
            pltpu.VMEM((_GRP, _K), jnp.int32),
            pltpu.VMEM((_GRP, _K), jnp.int32),
            pltpu.VMEM((2, _K, _H), jnp.float32),
            pltpu.VMEM_SHARED((_NPAD, _H), jnp.float32),
            pltpu.SemaphoreType.DMA,
            pltpu.SemaphoreType.DMA,
        ],
    )
    def body(h_hbm, ls_hbm, ld_hbm, cnt_hbm, zeros_hbm, out_hbm,
             cnt_v, sidx, didx, rows_v, acc_sh, sem0, sem1):
        cid = lax.axis_index("c")
        sid = lax.axis_index("s")
        w = cid * _NS + sid
        r0 = w * _STRIPE

        pltpu.sync_copy(zeros_hbm.at[pl.ds(pl.multiple_of(r0, 8), _STRIPE)],
                        acc_sh.at[pl.ds(pl.multiple_of(r0, 8), _STRIPE)])
        pltpu.sync_copy(cnt_hbm.at[pl.ds(pl.multiple_of(w * 16, 16), 16)],
                        cnt_v)
        ngrp = cnt_v[...][0] // _GRP
        sems = (sem0, sem1)

        def group(g, carry):
            goff = pl.multiple_of(g * _GRP, _GRP)
            pltpu.sync_copy(ls_hbm.at[w, pl.ds(goff, _GRP)], sidx)
            pltpu.sync_copy(ld_hbm.at[w, pl.ds(goff, _GRP)], didx)
            descs = [None, None]
            descs[0] = pltpu.async_copy(h_hbm.at[sidx.at[0]], rows_v.at[0],
                                        sems[0])
            for b in range(_GRP):
                if b + 1 < _GRP:
                    descs[(b + 1) % 2] = pltpu.async_copy(
                        h_hbm.at[sidx.at[b + 1]], rows_v.at[(b + 1) % 2],
                        sems[(b + 1) % 2])
                descs[b % 2].wait()
                pltpu.sync_copy(rows_v.at[b % 2], acc_sh.at[didx.at[b]],
                                add=True)
            return carry

        lax.fori_loop(0, ngrp, group, 0)

        pltpu.sync_copy(acc_sh.at[pl.ds(pl.multiple_of(r0, 8), _STRIPE)],
                        out_hbm.at[pl.ds(pl.multiple_of(r0, 8), _STRIPE)])

    return body(h, ls3, ld3, counts, zeros)


_B = 2000


def _encoder(z_r, pos8, embed, pos_w8, pos_b, comb_w, comb_b):
    nb = _N // _B

    def body(z_ref, pos_ref, emb_ref, pw_ref, pb_ref, cw_ref, cb_ref, out_ref):
        zb = z_ref[0, 0, :]
        onehot = (zb[:, None] == lax.broadcasted_iota(jnp.int32, (1, _NZ), 1)
                  ).astype(jnp.float32)
        atom = jnp.dot(onehot, emb_ref[...],
                       preferred_element_type=jnp.float32)
        pe = jnp.dot(pos_ref[...], pw_ref[...],
                     preferred_element_type=jnp.float32) + pb_ref[...][None, :]
        cat = jnp.concatenate([atom, pe], axis=1)
        acc = jnp.dot(cat, cw_ref[...],
                      preferred_element_type=jnp.float32) + cb_ref[...][None, :]
        out_ref[...] = jnp.maximum(acc, 0.0)

    return pl.pallas_call(
        body,
        grid=(nb,),
        in_specs=[
            pl.BlockSpec((1, 1, _B), lambda i: (i, 0, 0)),
            pl.BlockSpec((_B, 8), lambda i: (i, 0)),
            pl.BlockSpec((_NZ, _H), lambda i: (0, 0)),
            pl.BlockSpec((8, _H), lambda i: (0, 0)),
            pl.BlockSpec((_H,), lambda i: (0,)),
            pl.BlockSpec((2 * _H, _H), lambda i: (0, 0)),
            pl.BlockSpec((_H,), lambda i: (0,)),
        ],
        out_specs=pl.BlockSpec((_B, _H), lambda i: (i, 0)),
        out_shape=jax.ShapeDtypeStruct((_N, _H), jnp.float32),
    )(z_r, pos8, embed, pos_w8, pos_b, comb_w, comb_b)


def _gin_layer(h, msg, w1, b1, w2, b2, relu_out):
    nb = _N // _B
    d_out = w1.shape[1]

    def body(h_ref, m_ref, w1_ref, b1_ref, w2_ref, b2_ref, out_ref):
        a = h_ref[...] + m_ref[...]
        t = jnp.dot(a, w1_ref[...],
                    preferred_element_type=jnp.float32) + b1_ref[...][None, :]
        t = jnp.maximum(t, 0.0)
        o = jnp.dot(t, w2_ref[...],
                    preferred_element_type=jnp.float32) + b2_ref[...][None, :]
        out_ref[...] = jnp.maximum(o, 0.0) if relu_out else o

    return pl.pallas_call(
        body,
        grid=(nb,),
        in_specs=[
            pl.BlockSpec((_B, _H), lambda i: (i, 0)),
            pl.BlockSpec((_B, _H), lambda i: (i, 0)),
            pl.BlockSpec((_H, d_out), lambda i: (0, 0)),
            pl.BlockSpec((d_out,), lambda i: (0,)),
            pl.BlockSpec((d_out, d_out), lambda i: (0, 0)),
            pl.BlockSpec((d_out,), lambda i: (0,)),
        ],
        out_specs=pl.BlockSpec((_B, d_out), lambda i: (i, 0)),
        out_shape=jax.ShapeDtypeStruct((_N, d_out), jnp.float32),
    )(h, msg, w1, b1, w2, b2)


def _pool(batch_r, h):
    nb = _N // _B

    def body(b_ref, h_ref, out_ref):
        i = pl.program_id(0)
        bb = b_ref[0, 0, :]
        onehot = (bb[:, None] == lax.broadcasted_iota(jnp.int32, (1, _G), 1)
                  ).astype(jnp.float32)
        contrib = lax.dot_general(
            onehot, h_ref[...], (((0,), (0,)), ((), ())),
            precision=_PREC, preferred_element_type=jnp.float32)

        @pl.when(i == 0)
        def _():
            out_ref[...] = contrib

        @pl.when(i > 0)
        def _():
            out_ref[...] += contrib

    return pl.pallas_call(
        body,
        grid=(nb,),
        in_specs=[
            pl.BlockSpec((1, 1, _B), lambda i: (i, 0, 0)),
            pl.BlockSpec((_B, _OUT), lambda i: (i, 0)),
        ],
        out_specs=pl.BlockSpec((_G, _OUT), lambda i: (0, 0)),
        out_shape=jax.ShapeDtypeStruct((_G, _OUT), jnp.float32),
    )(batch_r, h)


def _heads(aggr, hp):
    def body(g_ref,
             aw1, ab1, aw2, ab2,
             bw1, bb1, bw2, bb2,
             nw1, nb1, nw2, nb2,
             gw1, gb1, gw2, gb2,
             gamma_ref, alea_ref, epis_ref, nu_ref, alpha_ref, beta_ref):
        g = g_ref[...]

        def head(w1, b1, w2, b2):
            a = jnp.dot(g, w1[...],
                        preferred_element_type=jnp.float32) + b1[...][None, :]
            a = jnp.maximum(a, 0.0)
            return (jnp.dot(a, w2[...],
                            preferred_element_type=jnp.float32)
                    + b2[...][None, :])

        s_alpha = head(aw1, ab1, aw2, ab2)
        s_beta = head(bw1, bb1, bw2, bb2)
        s_nu = head(nw1, nb1, nw2, nb2)
        s_gamma = head(gw1, gb1, gw2, gb2)

        nu = jax.nn.softplus(s_nu)
        alpha = jnp.maximum(jax.nn.softplus(s_alpha) + 1.0, 1.0 + 1e-4)
        beta = jax.nn.softplus(s_beta)
        gamma_ref[...] = s_gamma
        alea_ref[...] = beta / (alpha - 1.0)
        epis_ref[...] = beta / ((alpha - 1.0) * nu)
        nu_ref[...] = nu
        alpha_ref[...] = alpha
        beta_ref[...] = beta

    args = [aggr]
    for name in ["alpha", "beta", "nu", "gamma"]:
        p = hp[name]
        args += [p["W1"], p["b1"], p["W2"], p["b2"]]
    out = pl.pallas_call(
        body,
        out_shape=[jax.ShapeDtypeStruct((_G, 1), jnp.float32)] * 6,
    )(*args)
    return tuple(out)


def kernel(z, pos, edge_index, batch, params):
    z = z.astype(jnp.int32)
    src = edge_index[0].astype(jnp.int32)
    dst = edge_index[1].astype(jnp.int32)
    batch = batch.astype(jnp.int32)

    zpad = jnp.zeros((_ESCH - _E,), jnp.int32)
    spad = jnp.full((_ESCH - _E,), _SENT, jnp.int32)
    scan_src = jnp.concatenate([src, zpad])
    scan_dst = jnp.concatenate([dst, spad])
    lists_src, lists_dst, counts = _bucket_edges(scan_src, scan_dst)
    zeros = jnp.zeros((_NPAD, _H), jnp.float32)

    z_r = z.reshape(_N // _B, 1, _B)
    batch_r = batch.reshape(_N // _B, 1, _B)
    pos8 = jnp.pad(pos, ((0, 0), (0, 8 - pos.shape[1])))
    pos_w8 = jnp.pad(params["pos_W"], ((0, 8 - params["pos_W"].shape[0]), (0, 0)))

    h = _encoder(z_r, pos8, params["embed"], pos_w8, params["pos_b"],
                 params["comb_W"], params["comb_b"])

    n_layers = len(params["gin"])
    for i, lyr in enumerate(params["gin"]):
        msg = _edge_segment_sum(h, lists_src, lists_dst, counts, zeros)
        h = _gin_layer(h, msg, lyr["W1"], lyr["b1"], lyr["W2"], lyr["b2"],
                       relu_out=(i < n_layers - 1))

    aggr = _pool(batch_r, h)
    return _heads(aggr, params["heads"])


# speedup vs baseline: 2.4332x; 2.4332x over previous
import functools

import jax
import jax.numpy as jnp
from jax import lax
from jax.experimental import pallas as pl
from jax.experimental.pallas import tpu as pltpu
from jax.experimental.pallas import tpu_sc as plsc

_N = 10000
_E = 320000
_H = 128
_OUT = 128
_G = 64
_NZ = 100

_NC = 2
_NS = 16
_NW = _NC * _NS
_K = 128
_NPAD = -(-_N // (_NW * 8)) * (_NW * 8)
_STRIPE = _NPAD // _NW

_SCH = 2048
_NSC = -(-_E // _SCH)
_ESCH = _NSC * _SCH
_SENT = 0x3FFFFFFF
_FLUSH = 2048
_GRP = 8
_TAIL = _FLUSH + _GRP * _K
_BUF = 4160
_CAP2 = 323072

_PREC = lax.Precision.HIGHEST


def _bucket_edges(scan_src, scan_dst):
    @functools.partial(
        pl.kernel,
        out_type=(jax.ShapeDtypeStruct((_NW * _CAP2,), jnp.int32),
                  jax.ShapeDtypeStruct((_NW * _CAP2,), jnp.int32),
                  jax.ShapeDtypeStruct((_NW * 16,), jnp.int32)),
        mesh=plsc.VectorSubcoreMesh(core_axis_name="c", subcore_axis_name="s"),
        compiler_params=pltpu.CompilerParams(needs_layout_passes=False),
        scratch_types=[
            pltpu.VMEM((_SCH,), jnp.int32),
            pltpu.VMEM((_SCH,), jnp.int32),
            pltpu.VMEM((_BUF,), jnp.int32),
            pltpu.VMEM((_BUF,), jnp.int32),
            pltpu.VMEM((16,), jnp.int32),
        ],
    )
    def body(ss_hbm, sd_hbm, ls_hbm, ld_hbm, cnt_hbm,
             scan_s, scan_d, buf_s, buf_d, cnt_v):
        cid = lax.axis_index("c")
        sid = lax.axis_index("s")
        w = cid * _NS + sid
        lo = w * _STRIPE
        hi = lo + _STRIPE
        base = w * _CAP2

        def scan_chunk(ch, carry):
            fill, cnt = carry
            e0 = ch * _SCH
            pltpu.sync_copy(ss_hbm.at[pl.ds(pl.multiple_of(e0, _SCH), _SCH)],
                            scan_s)
            pltpu.sync_copy(sd_hbm.at[pl.ds(pl.multiple_of(e0, _SCH), _SCH)],
                            scan_d)

            def vec(i, fill):
                vs = scan_s[pl.ds(i * 16, 16)]
                vd = scan_d[pl.ds(i * 16, 16)]
                m = jnp.logical_and(vd >= lo, vd < hi)
                mi = jnp.where(m, jnp.int32(1), jnp.int32(0))
                incl = plsc.cumsum(mi)
                idx = (incl - mi) + fill
                plsc.store_scatter(buf_s, [idx], vs, mask=m)
                plsc.store_scatter(buf_d, [idx], vd, mask=m)
                return fill + incl[15]

            fill = lax.fori_loop(0, _SCH // 16, vec, fill, unroll=4)
            do_flush = fill >= _FLUSH

            @pl.when(do_flush)
            def _():
                off = pl.multiple_of(base + cnt, _K)
                pltpu.sync_copy(buf_s.at[pl.ds(0, _FLUSH)],
                                ls_hbm.at[pl.ds(off, _FLUSH)])
                pltpu.sync_copy(buf_d.at[pl.ds(0, _FLUSH)],
                                ld_hbm.at[pl.ds(off, _FLUSH)])
                rem = fill - _FLUSH

                def mv(k, c):
                    buf_s[pl.ds(k * 16, 16)] = buf_s[pl.ds(_FLUSH + k * 16, 16)]
                    buf_d[pl.ds(k * 16, 16)] = buf_d[pl.ds(_FLUSH + k * 16, 16)]
                    return c

                lax.fori_loop(0, (rem + 15) // 16, mv, 0)

            fill = jnp.where(do_flush, fill - _FLUSH, fill)
            cnt = jnp.where(do_flush, cnt + _FLUSH, cnt)
            return fill, cnt

        fill, cnt = lax.fori_loop(0, _NSC, scan_chunk,
                                  (jnp.int32(0), jnp.int32(0)))

        zvec = jnp.zeros((16,), jnp.int32)
        dvec = jnp.full((16,), _N, jnp.int32)
        for t in range(_GRP * _K // 16 + 1):
            buf_s[pl.ds(fill + t * 16, 16)] = zvec
            buf_d[pl.ds(fill + t * 16, 16)] = dvec
        off = pl.multiple_of(base + cnt, _K)
        pltpu.sync_copy(buf_s.at[pl.ds(0, _TAIL)], ls_hbm.at[pl.ds(off, _TAIL)])
        pltpu.sync_copy(buf_d.at[pl.ds(0, _TAIL)], ld_hbm.at[pl.ds(off, _TAIL)])

        n_ch = cnt // _K + (fill + _K - 1) // _K
        cnt_v[...] = jnp.full((16,), 1, jnp.int32) * n_ch
        pltpu.sync_copy(cnt_v,
                        cnt_hbm.at[pl.ds(pl.multiple_of(w * 16, 16), 16)])

    return body(scan_src, scan_dst)


def _edge_segment_sum(h, lists_src, lists_dst, counts, zeros):
    @functools.partial(
        pl.kernel,
        out_type=jax.ShapeDtypeStruct((_NPAD, _H), jnp.float32),
        mesh=plsc.VectorSubcoreMesh(core_axis_name="c", subcore_axis_name="s"),
        compiler_params=pltpu.CompilerParams(needs_layout_passes=False),
        scratch_types=[
            pltpu.VMEM((16,), jnp.int32),
            pltpu.VMEM((2, _K), jnp.int32),
            pltpu.VMEM((2, _K), jnp.int32),
            pltpu.VMEM((2, _K, _H), jnp.float32),
            pltpu.VMEM_SHARED((_NPAD, _H), jnp.float32),
            pltpu.SemaphoreType.DMA,
        ],
    )
    def body(h_hbm, ls_hbm, ld_hbm, cnt_hbm, zeros_hbm, out_hbm,
             cnt_v, sidx, didx, rows_v, acc_sh, sem):
        cid = lax.axis_index("c")
        sid = lax.axis_index("s")
        w = cid * _NS + sid
        base = w * _CAP2
        r0 = w * _STRIPE

        pltpu.sync_copy(zeros_hbm.at[pl.ds(pl.multiple_of(r0, 8), _STRIPE)],
                        acc_sh.at[pl.ds(pl.multiple_of(r0, 8), _STRIPE)])
        pltpu.sync_copy(cnt_hbm.at[pl.ds(pl.multiple_of(w * 16, 16), 16)],
                        cnt_v)
        nch = cnt_v[...][0]

        def stage(j, p):
            off = pl.multiple_of(base + j * _K, _K)
            pltpu.sync_copy(ls_hbm.at[pl.ds(off, _K)], sidx.at[p])
            pltpu.sync_copy(ld_hbm.at[pl.ds(off, _K)], didx.at[p])
            pltpu.async_copy(h_hbm.at[sidx.at[p]], rows_v.at[p], sem)

        @pl.when(nch > 0)
        def _():
            stage(jnp.int32(0), jnp.int32(0))

        def step(j, carry):
            p = j % 2

            @pl.when(j + 1 < nch)
            def _():
                stage(j + 1, 1 - p)

            pltpu.make_async_copy(h_hbm.at[pl.ds(0, _K)], rows_v.at[p],
                                  sem).wait()
            pltpu.sync_copy(rows_v.at[p], acc_sh.at[didx.at[p]], add=True)
            return carry

        lax.fori_loop(0, nch, step, 0)

        pltpu.sync_copy(acc_sh.at[pl.ds(pl.multiple_of(r0, 8), _STRIPE)],
                        out_hbm.at[pl.ds(pl.multiple_of(r0, 8), _STRIPE)])

    return body(h, lists_src, lists_dst, counts, zeros)


_B = 2000


def _encoder(z_r, pos8, embed, pos_w8, pos_b, comb_w, comb_b):
    nb = _N // _B

    def body(z_ref, pos_ref, emb_ref, pw_ref, pb_ref, cw_ref, cb_ref, out_ref):
        zb = z_ref[0, 0, :]
        onehot = (zb[:, None] == lax.broadcasted_iota(jnp.int32, (1, _NZ), 1)
                  ).astype(jnp.float32)
        atom = jnp.dot(onehot, emb_ref[...],
                       preferred_element_type=jnp.float32)
        pe = jnp.dot(pos_ref[...], pw_ref[...],
                     preferred_element_type=jnp.float32) + pb_ref[...][None, :]
        cat = jnp.concatenate([atom, pe], axis=1)
        acc = jnp.dot(cat, cw_ref[...],
                      preferred_element_type=jnp.float32) + cb_ref[...][None, :]
        out_ref[...] = jnp.maximum(acc, 0.0)

    return pl.pallas_call(
        body,
        grid=(nb,),
        in_specs=[
            pl.BlockSpec((1, 1, _B), lambda i: (i, 0, 0)),
            pl.BlockSpec((_B, 8), lambda i: (i, 0)),
            pl.BlockSpec((_NZ, _H), lambda i: (0, 0)),
            pl.BlockSpec((8, _H), lambda i: (0, 0)),
            pl.BlockSpec((_H,), lambda i: (0,)),
            pl.BlockSpec((2 * _H, _H), lambda i: (0, 0)),
            pl.BlockSpec((_H,), lambda i: (0,)),
        ],
        out_specs=pl.BlockSpec((_B, _H), lambda i: (i, 0)),
        out_shape=jax.ShapeDtypeStruct((_N, _H), jnp.float32),
    )(z_r, pos8, embed, pos_w8, pos_b, comb_w, comb_b)


def _gin_layer(h, msg, w1, b1, w2, b2, relu_out):
    nb = _N // _B
    d_out = w1.shape[1]

    def body(h_ref, m_ref, w1_ref, b1_ref, w2_ref, b2_ref, out_ref):
        a = h_ref[...] + m_ref[...]
        t = jnp.dot(a, w1_ref[...],
                    preferred_element_type=jnp.float32) + b1_ref[...][None, :]
        t = jnp.maximum(t, 0.0)
        o = jnp.dot(t, w2_ref[...],
                    preferred_element_type=jnp.float32) + b2_ref[...][None, :]
        out_ref[...] = jnp.maximum(o, 0.0) if relu_out else o

    return pl.pallas_call(
        body,
        grid=(nb,),
        in_specs=[
            pl.BlockSpec((_B, _H), lambda i: (i, 0)),
            pl.BlockSpec((_B, _H), lambda i: (i, 0)),
            pl.BlockSpec((_H, d_out), lambda i: (0, 0)),
            pl.BlockSpec((d_out,), lambda i: (0,)),
            pl.BlockSpec((d_out, d_out), lambda i: (0, 0)),
            pl.BlockSpec((d_out,), lambda i: (0,)),
        ],
        out_specs=pl.BlockSpec((_B, d_out), lambda i: (i, 0)),
        out_shape=jax.ShapeDtypeStruct((_N, d_out), jnp.float32),
    )(h, msg, w1, b1, w2, b2)


def _pool(batch_r, h):
    nb = _N // _B

    def body(b_ref, h_ref, out_ref):
        i = pl.program_id(0)
        bb = b_ref[0, 0, :]
        onehot = (bb[:, None] == lax.broadcasted_iota(jnp.int32, (1, _G), 1)
                  ).astype(jnp.float32)
        contrib = lax.dot_general(
            onehot, h_ref[...], (((0,), (0,)), ((), ())),
            precision=_PREC, preferred_element_type=jnp.float32)

        @pl.when(i == 0)
        def _():
            out_ref[...] = contrib

        @pl.when(i > 0)
        def _():
            out_ref[...] += contrib

    return pl.pallas_call(
        body,
        grid=(nb,),
        in_specs=[
            pl.BlockSpec((1, 1, _B), lambda i: (i, 0, 0)),
            pl.BlockSpec((_B, _OUT), lambda i: (i, 0)),
        ],
        out_specs=pl.BlockSpec((_G, _OUT), lambda i: (0, 0)),
        out_shape=jax.ShapeDtypeStruct((_G, _OUT), jnp.float32),
    )(batch_r, h)


def _heads(aggr, hp):
    def body(g_ref,
             aw1, ab1, aw2, ab2,
             bw1, bb1, bw2, bb2,
             nw1, nb1, nw2, nb2,
             gw1, gb1, gw2, gb2,
             gamma_ref, alea_ref, epis_ref, nu_ref, alpha_ref, beta_ref):
        g = g_ref[...]

        def head(w1, b1, w2, b2):
            a = jnp.dot(g, w1[...],
                        preferred_element_type=jnp.float32) + b1[...][None, :]
            a = jnp.maximum(a, 0.0)
            return (jnp.dot(a, w2[...],
                            preferred_element_type=jnp.float32)
                    + b2[...][None, :])

        s_alpha = head(aw1, ab1, aw2, ab2)
        s_beta = head(bw1, bb1, bw2, bb2)
        s_nu = head(nw1, nb1, nw2, nb2)
        s_gamma = head(gw1, gb1, gw2, gb2)

        nu = jax.nn.softplus(s_nu)
        alpha = jnp.maximum(jax.nn.softplus(s_alpha) + 1.0, 1.0 + 1e-4)
        beta = jax.nn.softplus(s_beta)
        gamma_ref[...] = s_gamma
        alea_ref[...] = beta / (alpha - 1.0)
        epis_ref[...] = beta / ((alpha - 1.0) * nu)
        nu_ref[...] = nu
        alpha_ref[...] = alpha
        beta_ref[...] = beta

    args = [aggr]
    for name in ["alpha", "beta", "nu", "gamma"]:
        p = hp[name]
        args += [p["W1"], p["b1"], p["W2"], p["b2"]]
    out = pl.pallas_call(
        body,
        out_shape=[jax.ShapeDtypeStruct((_G, 1), jnp.float32)] * 6,
    )(*args)
    return tuple(out)


def kernel(z, pos, edge_index, batch, params):
    z = z.astype(jnp.int32)
    src = edge_index[0].astype(jnp.int32)
    dst = edge_index[1].astype(jnp.int32)
    batch = batch.astype(jnp.int32)

    zpad = jnp.zeros((_ESCH - _E,), jnp.int32)
    spad = jnp.full((_ESCH - _E,), _SENT, jnp.int32)
    scan_src = jnp.concatenate([src, zpad])
    scan_dst = jnp.concatenate([dst, spad])
    lists_src, lists_dst, counts = _bucket_edges(scan_src, scan_dst)
    zeros = jnp.zeros((_NPAD, _H), jnp.float32)

    z_r = z.reshape(_N // _B, 1, _B)
    batch_r = batch.reshape(_N // _B, 1, _B)
    pos8 = jnp.pad(pos, ((0, 0), (0, 8 - pos.shape[1])))
    pos_w8 = jnp.pad(params["pos_W"], ((0, 8 - params["pos_W"].shape[0]), (0, 0)))

    h = _encoder(z_r, pos8, params["embed"], pos_w8, params["pos_b"],
                 params["comb_W"], params["comb_b"])

    n_layers = len(params["gin"])
    for i, lyr in enumerate(params["gin"]):
        msg = _edge_segment_sum(h, lists_src, lists_dst, counts, zeros)
        h = _gin_layer(h, msg, lyr["W1"], lyr["b1"], lyr["W2"], lyr["b2"],
                       relu_out=(i < n_layers - 1))

    aggr = _pool(batch_r, h)
    return _heads(aggr, params["heads"])
